# Initial kernel scaffold; baseline (speedup 1.0000x reference)
import functools

import jax
import jax.numpy as jnp
from jax import lax
from jax.experimental import pallas as pl
from jax.experimental.pallas import tpu as pltpu
from jax.experimental.pallas import tpu_sc as plsc

_N = 10000
_E = 320000
_NW = 32
_NCH = 80
_CH = 125
_RPT = _N // 16

_HIGH = lax.Precision.HIGHEST


def _spmm_partial(feat, src3, dst3):
    F = feat.shape[1]
    mesh = plsc.VectorSubcoreMesh(core_axis_name="c", subcore_axis_name="s")

    @functools.partial(
        pl.kernel,
        mesh=mesh,
        out_type=jax.ShapeDtypeStruct((2, _N, F), jnp.float32),
        scratch_types=[
            pltpu.VMEM((_NCH, _CH), jnp.int32),
            pltpu.VMEM((_NCH, _CH), jnp.int32),
            pltpu.VMEM((_CH, F), jnp.float32),
            pltpu.VMEM((_CH, F), jnp.float32),
            pltpu.VMEM_SHARED((_N, F), jnp.float32),
            pltpu.SemaphoreType.DMA,
            pltpu.SemaphoreType.DMA,
        ],
    )
    def spmm(feat_hbm, src_hbm, dst_hbm, out_hbm,
             src_v, dst_v, buf0, buf1, acc, sem0, sem1):
        c = lax.axis_index("c")
        s = lax.axis_index("s")
        w = c * 16 + s

        @pl.loop(0, _CH)
        def _(i):
            @pl.loop(0, F // 16)
            def _(j):
                buf0[i, pl.ds(j * 16, 16)] = jnp.zeros((16,), jnp.float32)

        @pl.loop(0, _RPT // _CH)
        def _(t):
            pltpu.sync_copy(buf0, acc.at[pl.ds(s * _RPT + t * _CH, _CH)])

        plsc.subcore_barrier()

        pltpu.sync_copy(src_hbm.at[w], src_v)
        pltpu.sync_copy(dst_hbm.at[w], dst_v)

        def start(j, buf, sem):
            pltpu.async_copy(feat_hbm.at[src_v.at[j]], buf, sem)

        def wait(buf, sem):
            pltpu.make_async_copy(feat_hbm.at[src_v.at[0]], buf, sem).wait()

        start(0, buf0, sem0)
        start(1, buf1, sem1)

        @pl.loop(0, _NCH - 2, step=2)
        def _(j):
            wait(buf0, sem0)
            pltpu.sync_copy(buf0, acc.at[dst_v.at[j]], add=True)
            start(j + 2, buf0, sem0)
            wait(buf1, sem1)
            pltpu.sync_copy(buf1, acc.at[dst_v.at[j + 1]], add=True)
            start(j + 3, buf1, sem1)

        wait(buf0, sem0)
        pltpu.sync_copy(buf0, acc.at[dst_v.at[_NCH - 2]], add=True)
        wait(buf1, sem1)
        pltpu.sync_copy(buf1, acc.at[dst_v.at[_NCH - 1]], add=True)

        plsc.subcore_barrier()

        @pl.loop(0, _RPT // _CH)
        def _(t):
            off = s * _RPT + t * _CH
            pltpu.sync_copy(acc.at[pl.ds(off, _CH)],
                            out_hbm.at[c].at[pl.ds(off, _CH)])

    return spmm(feat, src3, dst3)


_BM = 1000


def _leaky(v):
    return jnp.where(v >= 0, v, v * jnp.float32(0.01))


def _sigmoid(v):
    return 1.0 / (1.0 + jnp.exp(-v))


def _softplus_clip(v):
    sp = jnp.maximum(v, 0.0) + jnp.log1p(jnp.exp(-jnp.abs(v)))
    return jnp.clip(sp, 1e-05, 1000000.0)


def _exp_clip(v):
    return jnp.clip(jnp.exp(v), 1e-05, 1000000.0)


def _mm(a, w):
    m, k = a.shape
    f = w.shape[1]

    def body(a_ref, w_ref, o_ref):
        o_ref[...] = jnp.dot(a_ref[...], w_ref[...],
                             preferred_element_type=jnp.float32,
                             precision=_HIGH)

    return pl.pallas_call(
        body,
        grid=(m // _BM,),
        in_specs=[pl.BlockSpec((_BM, k), lambda i: (i, 0)),
                  pl.BlockSpec((k, f), lambda i: (0, 0))],
        out_specs=pl.BlockSpec((_BM, f), lambda i: (i, 0)),
        out_shape=jax.ShapeDtypeStruct((m, f), jnp.float32),
    )(a, w)


def _heads(p, specs):
    _, m, k = p.shape
    n = len(specs)

    in_specs = [pl.BlockSpec((2, _BM, k), lambda i: (0, i, 0))]
    operands = [p]
    out_shapes = []
    out_specs = []
    for wgt, b, _ in specs:
        f = k if wgt is None else wgt.shape[1]
        if wgt is not None:
            in_specs.append(pl.BlockSpec(wgt.shape, lambda i: (0, 0)))
            operands.append(wgt)
        in_specs.append(pl.BlockSpec((1, f), lambda i: (0, 0)))
        operands.append(b.reshape(1, f))
        out_shapes.append(jax.ShapeDtypeStruct((m, f), jnp.float32))
        out_specs.append(pl.BlockSpec((_BM, f), lambda i: (i, 0)))

    def body(p_ref, *refs):
        agg = p_ref[0] + p_ref[1]
        o_refs = refs[len(refs) - n:]
        pos = 0
        for (wgt, _, act), o_ref in zip(specs, o_refs):
            if wgt is None:
                v = agg + refs[pos][...]
                pos += 1
            else:
                v = jnp.dot(agg, refs[pos][...],
                            preferred_element_type=jnp.float32,
                            precision=_HIGH) + refs[pos + 1][...]
                pos += 2
            o_ref[...] = act(v)

    outs = pl.pallas_call(
        body,
        grid=(m // _BM,),
        in_specs=in_specs,
        out_specs=out_specs,
        out_shape=out_shapes,
    )(*operands)
    return outs if n > 1 else (outs,)


def _inner_product(z, zt):
    m, k = z.shape
    bn = 2000

    def body(a_ref, b_ref, o_ref):
        o_ref[...] = jnp.dot(a_ref[...], b_ref[...],
                             preferred_element_type=jnp.float32,
                             precision=_HIGH)

    return pl.pallas_call(
        body,
        grid=(m // _BM, m // bn),
        in_specs=[pl.BlockSpec((_BM, k), lambda i, j: (i, 0)),
                  pl.BlockSpec((k, bn), lambda i, j: (0, j))],
        out_specs=pl.BlockSpec((_BM, bn), lambda i, j: (i, j)),
        out_shape=jax.ShapeDtypeStruct((m, m), jnp.float32),
    )(z, zt)


def kernel(x, edge_index, W1, b1, W2, b2, W2s, b2s,
           Wd1, bd1, Wpi, bpi, Wth, bth, Wmn, bmn):
    src3 = edge_index[0].reshape(_NW, _NCH, _CH)
    dst3 = edge_index[1].reshape(_NW, _NCH, _CH)

    s1 = _mm(x, W1)
    p1 = _spmm_partial(s1, src3, dst3)
    (h1,) = _heads(p1, [(None, b1, _leaky)])

    q = _spmm_partial(h1, src3, dst3)
    mu, logvar = _heads(q, [(W2, b2, _leaky), (W2s, b2s, _leaky)])
    z = mu

    r = _spmm_partial(z, src3, dst3)
    (dec_out,) = _heads(r, [(Wd1, bd1, _leaky)])

    sagg = _spmm_partial(dec_out, src3, dst3)
    pi_res, theta_res, mean_res = _heads(
        sagg,
        [(Wpi, bpi, _sigmoid), (Wth, bth, _softplus_clip), (Wmn, bmn, _exp_clip)])

    dc_out = _inner_product(z, z.T)

    return (dc_out, mu, logvar, z, dec_out, pi_res, theta_res, mean_res)


# speedup vs baseline: 6.9457x; 6.9457x over previous
import functools

import jax
import jax.numpy as jnp
from jax import lax
from jax.experimental import pallas as pl
from jax.experimental.pallas import tpu as pltpu
from jax.experimental.pallas import tpu_sc as plsc

_N = 10000
_E = 320000
_NW = 32
_NCH = 100
_CH = 100
_RPT = _N // 16


def _spmm_partial(feat, src3, dst3):
    F = feat.shape[1]
    H = F // 64
    feats = [feat] if H == 1 else [feat[:, 64 * h:64 * (h + 1)] for h in range(H)]
    mesh = plsc.VectorSubcoreMesh(core_axis_name="c", subcore_axis_name="s")

    @functools.partial(
        pl.kernel,
        mesh=mesh,
        out_type=jax.ShapeDtypeStruct((2, H, _N, 64), jnp.float32),
        compiler_params=pltpu.CompilerParams(use_tc_tiling_on_sc=False),
        scratch_types=[
            pltpu.VMEM((_NCH, _CH), jnp.int32),
            pltpu.VMEM((_NCH, _CH), jnp.int32),
            pltpu.VMEM((_CH, 64), jnp.float32),
            pltpu.VMEM((_CH, 64), jnp.float32),
            pltpu.VMEM_SHARED((_N, 64), jnp.float32),
            pltpu.SemaphoreType.DMA,
            pltpu.SemaphoreType.DMA,
        ],
    )
    def spmm(*refs):
        feat_refs = refs[:H]
        src_hbm, dst_hbm, out_hbm = refs[H:H + 3]
        src_v, dst_v, buf0, buf1, acc, sem0, sem1 = refs[H + 3:]
        c = lax.axis_index("c")
        s = lax.axis_index("s")
        w = c * 16 + s

        pltpu.sync_copy(src_hbm.at[w], src_v)
        pltpu.sync_copy(dst_hbm.at[w], dst_v)

        for h in range(H):
            fh = feat_refs[h]

            @pl.loop(0, _CH)
            def _(i):
                @pl.loop(0, 4)
                def _(j):
                    buf0[i, pl.ds(j * 16, 16)] = jnp.zeros((16,), jnp.float32)

            @pl.loop(0, 6)
            def _(t):
                pltpu.sync_copy(buf0, acc.at[pl.ds(s * _RPT + t * _CH, _CH)])

            pltpu.sync_copy(buf0.at[pl.ds(0, _RPT - 6 * _CH)],
                            acc.at[pl.ds(s * _RPT + 6 * _CH, _RPT - 6 * _CH)])

            plsc.subcore_barrier()

            def start(j, buf, sem):
                pltpu.async_copy(fh.at[src_v.at[j]], buf, sem)

            def wait(buf, sem):
                pltpu.make_async_copy(fh.at[src_v.at[0]], buf, sem).wait()

            start(0, buf0, sem0)
            start(1, buf1, sem1)

            @pl.loop(0, _NCH - 2, step=2)
            def _(j):
                wait(buf0, sem0)
                pltpu.sync_copy(buf0, acc.at[dst_v.at[j]], add=True)
                start(j + 2, buf0, sem0)
                wait(buf1, sem1)
                pltpu.sync_copy(buf1, acc.at[dst_v.at[j + 1]], add=True)
                start(j + 3, buf1, sem1)

            wait(buf0, sem0)
            pltpu.sync_copy(buf0, acc.at[dst_v.at[_NCH - 2]], add=True)
            wait(buf1, sem1)
            pltpu.sync_copy(buf1, acc.at[dst_v.at[_NCH - 1]], add=True)

            plsc.subcore_barrier()

            wo = s * 624
            pltpu.sync_copy(acc.at[pl.ds(wo, 624)],
                            out_hbm.at[c].at[h].at[pl.ds(wo, 624)])

            @pl.when(s == 15)
            def _():
                pltpu.sync_copy(acc.at[pl.ds(9984, 16)],
                                out_hbm.at[c].at[h].at[pl.ds(9984, 16)])

            plsc.subcore_barrier()

    return spmm(*feats, src3, dst3)


_BM = 1000


def _bf_dot(a, b):
    return jnp.dot(a.astype(jnp.bfloat16), b.astype(jnp.bfloat16),
                   preferred_element_type=jnp.float32)


def _leaky(v):
    return jnp.where(v >= 0, v, v * jnp.float32(0.01))


def _sigmoid(v):
    return 1.0 / (1.0 + jnp.exp(-v))


def _softplus_clip(v):
    sp = jnp.maximum(v, 0.0) + jnp.log1p(jnp.exp(-jnp.abs(v)))
    return jnp.clip(sp, 1e-05, 1000000.0)


def _exp_clip(v):
    return jnp.clip(jnp.exp(v), 1e-05, 1000000.0)


def _mm(a, w):
    m, k = a.shape
    f = w.shape[1]

    def body(a_ref, w_ref, o_ref):
        o_ref[...] = _bf_dot(a_ref[...], w_ref[...])

    return pl.pallas_call(
        body,
        grid=(m // _BM,),
        in_specs=[pl.BlockSpec((_BM, k), lambda i: (i, 0)),
                  pl.BlockSpec((k, f), lambda i: (0, 0))],
        out_specs=pl.BlockSpec((_BM, f), lambda i: (i, 0)),
        out_shape=jax.ShapeDtypeStruct((m, f), jnp.float32),
    )(a, w)


def _heads(p, specs):
    _, hh, m, _ = p.shape
    n = len(specs)

    in_specs = [pl.BlockSpec((2, hh, _BM, 64), lambda i: (0, 0, i, 0))]
    operands = [p]
    out_shapes = []
    out_specs = []
    for hs, b, _ in specs:
        f = 64 * len(hs)
        in_specs.append(pl.BlockSpec((1, f), lambda i: (0, 0)))
        operands.append(b.reshape(1, f))
        out_shapes.append(jax.ShapeDtypeStruct((m, f), jnp.float32))
        out_specs.append(pl.BlockSpec((_BM, f), lambda i: (i, 0)))

    def body(p_ref, *refs):
        o_refs = refs[len(refs) - n:]
        for j, ((hs, _, act), o_ref) in enumerate(zip(specs, o_refs)):
            bb = refs[j][...]
            for t, h in enumerate(hs):
                sl = slice(t * 64, (t + 1) * 64)
                o_ref[:, sl] = act(p_ref[0, h] + p_ref[1, h] + bb[:, sl])

    outs = pl.pallas_call(
        body,
        grid=(m // _BM,),
        in_specs=in_specs,
        out_specs=out_specs,
        out_shape=out_shapes,
    )(*operands)
    return tuple(outs) if isinstance(outs, (list, tuple)) else (outs,)


def _inner_product(z, zt):
    m, k = z.shape
    bm = 400

    def body(a_ref, b_ref, o_ref):
        o_ref[...] = _bf_dot(a_ref[...], b_ref[...])

    return pl.pallas_call(
        body,
        grid=(m // bm,),
        in_specs=[pl.BlockSpec((bm, k), lambda i: (i, 0)),
                  pl.BlockSpec((k, m), lambda i: (0, 0))],
        out_specs=pl.BlockSpec((bm, m), lambda i: (i, 0)),
        out_shape=jax.ShapeDtypeStruct((m, m), jnp.float32),
    )(z, zt)


def kernel(x, edge_index, W1, b1, W2, b2, W2s, b2s,
           Wd1, bd1, Wpi, bpi, Wth, bth, Wmn, bmn):
    src3 = edge_index[0].reshape(_NW, _NCH, _CH)
    dst3 = edge_index[1].reshape(_NW, _NCH, _CH)

    sup1 = _mm(x, W1)
    p1 = _spmm_partial(sup1, src3, dst3)
    (h1,) = _heads(p1, [((0, 1), b1, _leaky)])

    sup2 = _mm(h1, jnp.concatenate([W2, W2s], axis=1))
    q = _spmm_partial(sup2, src3, dst3)
    mu, logvar = _heads(q, [((0,), b2, _leaky), ((1,), b2s, _leaky)])
    z = mu

    supd = _mm(z, Wd1)
    r = _spmm_partial(supd, src3, dst3)
    (dec_out,) = _heads(r, [((0, 1), bd1, _leaky)])

    supo = _mm(dec_out, jnp.concatenate([Wpi, Wth, Wmn], axis=1))
    sagg = _spmm_partial(supo, src3, dst3)
    pi_res, theta_res, mean_res = _heads(
        sagg,
        [(tuple(range(0, 4)), bpi, _sigmoid),
         (tuple(range(4, 8)), bth, _softplus_clip),
         (tuple(range(8, 12)), bmn, _exp_clip)])

    dc_out = _inner_product(z, z.T)

    return (dc_out, mu, logvar, z, dec_out, pi_res, theta_res, mean_res)


# speedup vs baseline: 8.7727x; 1.2630x over previous
import functools

import jax
import jax.numpy as jnp
from jax import lax
from jax.experimental import pallas as pl
from jax.experimental.pallas import tpu as pltpu
from jax.experimental.pallas import tpu_sc as plsc

_N = 10000
_E = 320000
_NW = 32
_NCH = 80
_CH = 125
_RPT = _N // 16


def _spmm_partial(feat, src3, dst3):
    F = feat.shape[1]
    H = F // 64
    feats = [feat] if H == 1 else [feat[:, 64 * h:64 * (h + 1)] for h in range(H)]
    mesh = plsc.VectorSubcoreMesh(core_axis_name="c", subcore_axis_name="s")

    @functools.partial(
        pl.kernel,
        mesh=mesh,
        out_type=jax.ShapeDtypeStruct((2, H, _N, 64), jnp.float32),
        compiler_params=pltpu.CompilerParams(use_tc_tiling_on_sc=False),
        scratch_types=[
            pltpu.VMEM((_NCH, _CH), jnp.int32),
            pltpu.VMEM((_NCH, _CH), jnp.int32),
            pltpu.VMEM((_CH, 64), jnp.float32),
            pltpu.VMEM((_CH, 64), jnp.float32),
            pltpu.VMEM((_CH, 64), jnp.float32),
            pltpu.VMEM((_CH, 64), jnp.float32),
            pltpu.VMEM_SHARED((_N, 64), jnp.float32),
            pltpu.SemaphoreType.DMA,
            pltpu.SemaphoreType.DMA,
            pltpu.SemaphoreType.DMA,
            pltpu.SemaphoreType.DMA,
        ],
    )
    def spmm(*refs):
        feat_refs = refs[:H]
        src_hbm, dst_hbm, out_hbm = refs[H:H + 3]
        src_v, dst_v = refs[H + 3:H + 5]
        bufs = refs[H + 5:H + 9]
        acc = refs[H + 9]
        sems = refs[H + 10:H + 14]
        c = lax.axis_index("c")
        s = lax.axis_index("s")
        w = c * 16 + s

        pltpu.sync_copy(src_hbm.at[w], src_v)
        pltpu.sync_copy(dst_hbm.at[w], dst_v)

        for h in range(H):
            fh = feat_refs[h]

            @pl.loop(0, _CH)
            def _(i):
                @pl.loop(0, 4)
                def _(j):
                    bufs[0][i, pl.ds(j * 16, 16)] = jnp.zeros((16,), jnp.float32)

            @pl.loop(0, _RPT // _CH)
            def _(t):
                pltpu.sync_copy(bufs[0], acc.at[pl.ds(s * _RPT + t * _CH, _CH)])

            plsc.subcore_barrier()

            def start(j, k):
                pltpu.async_copy(fh.at[src_v.at[j]], bufs[k], sems[k])

            def wait(k):
                pltpu.make_async_copy(fh.at[src_v.at[0]], bufs[k], sems[k]).wait()

            for k in range(4):
                start(k, k)

            @pl.loop(0, _NCH - 4, step=4)
            def _(j):
                for k in range(4):
                    wait(k)
                    pltpu.sync_copy(bufs[k], acc.at[dst_v.at[j + k]], add=True)
                    start(j + k + 4, k)

            for k in range(4):
                wait(k)
                pltpu.sync_copy(bufs[k], acc.at[dst_v.at[_NCH - 4 + k]], add=True)

            plsc.subcore_barrier()

            wo = s * 624
            pltpu.sync_copy(acc.at[pl.ds(wo, 624)],
                            out_hbm.at[c].at[h].at[pl.ds(wo, 624)])

            @pl.when(s == 15)
            def _():
                pltpu.sync_copy(acc.at[pl.ds(9984, 16)],
                                out_hbm.at[c].at[h].at[pl.ds(9984, 16)])

            plsc.subcore_barrier()

    return spmm(*feats, src3, dst3)


_BM = 1000


def _bf_dot(a, b):
    return jnp.dot(a.astype(jnp.bfloat16), b.astype(jnp.bfloat16),
                   preferred_element_type=jnp.float32)


def _leaky(v):
    return jnp.where(v >= 0, v, v * jnp.float32(0.01))


def _sigmoid(v):
    return 1.0 / (1.0 + jnp.exp(-v))


def _softplus_clip(v):
    sp = jnp.maximum(v, 0.0) + jnp.log1p(jnp.exp(-jnp.abs(v)))
    return jnp.clip(sp, 1e-05, 1000000.0)


def _exp_clip(v):
    return jnp.clip(jnp.exp(v), 1e-05, 1000000.0)


def _mm(a, w):
    m, k = a.shape
    f = w.shape[1]

    def body(a_ref, w_ref, o_ref):
        o_ref[...] = _bf_dot(a_ref[...], w_ref[...])

    return pl.pallas_call(
        body,
        grid=(m // _BM,),
        in_specs=[pl.BlockSpec((_BM, k), lambda i: (i, 0)),
                  pl.BlockSpec((k, f), lambda i: (0, 0))],
        out_specs=pl.BlockSpec((_BM, f), lambda i: (i, 0)),
        out_shape=jax.ShapeDtypeStruct((m, f), jnp.float32),
    )(a, w)


def _heads(p, specs):
    _, hh, m, _ = p.shape
    n = len(specs)

    in_specs = [pl.BlockSpec((2, hh, _BM, 64), lambda i: (0, 0, i, 0))]
    operands = [p]
    out_shapes = []
    out_specs = []
    for hs, b, _ in specs:
        f = 64 * len(hs)
        in_specs.append(pl.BlockSpec((1, f), lambda i: (0, 0)))
        operands.append(b.reshape(1, f))
        out_shapes.append(jax.ShapeDtypeStruct((m, f), jnp.float32))
        out_specs.append(pl.BlockSpec((_BM, f), lambda i: (i, 0)))

    def body(p_ref, *refs):
        o_refs = refs[len(refs) - n:]
        for j, ((hs, _, act), o_ref) in enumerate(zip(specs, o_refs)):
            bb = refs[j][...]
            for t, h in enumerate(hs):
                sl = slice(t * 64, (t + 1) * 64)
                o_ref[:, sl] = act(p_ref[0, h] + p_ref[1, h] + bb[:, sl])

    outs = pl.pallas_call(
        body,
        grid=(m // _BM,),
        in_specs=in_specs,
        out_specs=out_specs,
        out_shape=out_shapes,
    )(*operands)
    return tuple(outs) if isinstance(outs, (list, tuple)) else (outs,)


def _inner_product(z, zt):
    m, k = z.shape
    bm = 400

    def body(a_ref, b_ref, o_ref):
        o_ref[...] = _bf_dot(a_ref[...], b_ref[...])

    return pl.pallas_call(
        body,
        grid=(m // bm,),
        in_specs=[pl.BlockSpec((bm, k), lambda i: (i, 0)),
                  pl.BlockSpec((k, m), lambda i: (0, 0))],
        out_specs=pl.BlockSpec((bm, m), lambda i: (i, 0)),
        out_shape=jax.ShapeDtypeStruct((m, m), jnp.float32),
    )(z, zt)


def kernel(x, edge_index, W1, b1, W2, b2, W2s, b2s,
           Wd1, bd1, Wpi, bpi, Wth, bth, Wmn, bmn):
    src3 = edge_index[0].reshape(_NW, _NCH, _CH)
    dst3 = edge_index[1].reshape(_NW, _NCH, _CH)

    sup1 = _mm(x, W1)
    p1 = _spmm_partial(sup1, src3, dst3)
    (h1,) = _heads(p1, [((0, 1), b1, _leaky)])

    sup2 = _mm(h1, jnp.concatenate([W2, W2s], axis=1))
    q = _spmm_partial(sup2, src3, dst3)
    mu, logvar = _heads(q, [((0,), b2, _leaky), ((1,), b2s, _leaky)])
    z = mu

    supd = _mm(z, Wd1)
    r = _spmm_partial(supd, src3, dst3)
    (dec_out,) = _heads(r, [((0, 1), bd1, _leaky)])

    supo = _mm(dec_out, jnp.concatenate([Wpi, Wth, Wmn], axis=1))
    sagg = _spmm_partial(supo, src3, dst3)
    pi_res, theta_res, mean_res = _heads(
        sagg,
        [(tuple(range(0, 4)), bpi, _sigmoid),
         (tuple(range(4, 8)), bth, _softplus_clip),
         (tuple(range(8, 12)), bmn, _exp_clip)])

    dc_out = _inner_product(z, z.T)

    return (dc_out, mu, logvar, z, dec_out, pi_res, theta_res, mean_res)
